# baseline (speedup 1.0000x reference)
import functools

import jax
import jax.numpy as jnp
from jax.experimental import pallas as pl

B, N = 64, 24
NF, EF = 64, 4
HID, MSG, PASSES = 64, 32, 3

Q = 4
G = B // Q
GB = 8


def _blockdiag_mask(rows, cols, rblk, cblk):
    r = jax.lax.broadcasted_iota(jnp.int32, (rows, cols), 0)
    c = jax.lax.broadcasted_iota(jnp.int32, (rows, cols), 1)
    return (r // rblk == c // cblk).astype(jnp.float32)


def _mpnn_kernel(x_ref, e_ref, W_msg_ref, b_msg_ref, W_u_ref, W_m_ref,
                 b_u_ref, W_g_ref, W_o_ref, out_ref):
    f32 = jnp.float32
    NN = N * N
    QN = Q * N
    r_i = jax.lax.broadcasted_iota(jnp.int32, (N, NN), 1)
    n_i = jax.lax.broadcasted_iota(jnp.int32, (N, NN), 0)
    S0 = (r_i // N == n_i).astype(f32)
    r2_i = jax.lax.broadcasted_iota(jnp.int32, (NN, N), 0)
    g_i = jax.lax.broadcasted_iota(jnp.int32, (NN, N), 1)
    TileG = (r2_i % N == g_i).astype(f32)
    R = _blockdiag_mask(Q, QN, 1, N)

    W_h = W_msg_ref[0:HID, :]
    W_e = W_msg_ref[HID:HID + EF, :]
    b_msg = b_msg_ref[...]
    W_u = W_u_ref[...]
    W_m = W_m_ref[...]
    b_u = b_u_ref[...]
    W_gh = W_g_ref[0:HID, :]
    W_gx = W_g_ref[HID:2 * HID, :]
    W_oh = W_o_ref[0:HID, :]
    W_ox = W_o_ref[HID:2 * HID, :]

    WkH = jnp.tile(W_h, (Q, Q)) * _blockdiag_mask(Q * HID, Q * MSG, HID, MSG)
    WkE = jnp.tile(W_e, (Q, Q)) * _blockdiag_mask(Q * EF, Q * MSG, EF, MSG)
    _r = jax.lax.broadcasted_iota(jnp.int32, (Q * Q * MSG, 1), 0)
    WmP = jnp.tile(W_m, (Q * Q, 1)) * (
        (_r // (Q * MSG)) == ((_r % (Q * MSG)) // MSG)).astype(f32)
    b_msg4 = jnp.tile(b_msg, (1, Q))
    A32 = jnp.tile(jnp.ones((EF, MSG), f32), (Q, Q)) * _blockdiag_mask(
        Q * EF, Q * MSG, EF, MSG)

    dot = functools.partial(jnp.dot, preferred_element_type=f32)

    for gb in range(GB):
        xs = [x_ref[gb * Q + q] for q in range(Q)]
        es = [e_ref[gb * Q + q] for q in range(Q)]
        x_cat = jnp.concatenate(xs, axis=0)

        E4 = b_msg4
        adj32 = jnp.zeros((NN, Q * MSG), f32)
        for q in range(Q):
            E4 = E4 + dot(es[q], WkE[q * EF:(q + 1) * EF, :])
            adj32 = adj32 + dot(es[q], A32[q * EF:(q + 1) * EF, :])
        mask4 = (adj32 != 0.0).astype(f32)

        acts = []
        for q in range(Q):
            adj_q = jnp.sum(es[q], axis=1, keepdims=True)
            acts.append(dot(S0, adj_q))
        act = jnp.concatenate(acts, axis=0) != 0.0

        h_cat = x_cat
        for _ in range(PASSES):
            Hm4 = jnp.zeros((N, Q * MSG), f32)
            for q in range(Q):
                Hm4 = Hm4 + dot(h_cat[q * N:(q + 1) * N, :],
                                WkH[q * HID:(q + 1) * HID, :])
            T4 = jnp.tanh(E4 + dot(TileG, Hm4))
            msg4 = dot(S0, T4 * mask4)
            mm = jnp.concatenate(
                [dot(msg4, WmP[q * Q * MSG:(q + 1) * Q * MSG, :])
                 for q in range(Q)], axis=0)
            upd = jnp.tanh(dot(h_cat, W_u) + mm + b_u)
            h_cat = jnp.where(act, upd, h_cat)
        gate = jax.nn.sigmoid(dot(h_cat, W_gh) + dot(x_cat, W_gx))
        o = dot(h_cat, W_oh) + dot(x_cat, W_ox)
        gated = gate * o * act.astype(f32)
        out_ref[gb] = dot(R, gated)


def kernel(nodes, edges, W_msg, b_msg, W_u, W_m, b_u, W_g, W_o):
    f32 = jnp.float32
    eflat = edges.reshape(B, N * N, EF)
    b_msg2 = b_msg.reshape(1, MSG)
    b_u2 = b_u.reshape(1, HID)

    full = lambda shape: pl.BlockSpec(shape, lambda i: (0,) * len(shape))
    out = pl.pallas_call(
        _mpnn_kernel,
        grid=(G // GB,),
        in_specs=[
            pl.BlockSpec((GB * Q, N, NF), lambda i: (i, 0, 0)),
            pl.BlockSpec((GB * Q, N * N, EF), lambda i: (i, 0, 0)),
            full((HID + EF, MSG)),
            full((1, MSG)),
            full((HID, HID)),
            full((MSG, HID)),
            full((1, HID)),
            full((2 * HID, HID)),
            full((2 * HID, HID)),
        ],
        out_specs=pl.BlockSpec((GB, Q, HID), lambda i: (i, 0, 0)),
        out_shape=jax.ShapeDtypeStruct((G, Q, HID), f32),
    )(nodes, eflat, W_msg, b_msg2, W_u, W_m, b_u2, W_g, W_o)
    return out.reshape(B, HID)


# speedup vs baseline: 1.3946x; 1.3946x over previous
import functools

import jax
import jax.numpy as jnp
from jax.experimental import pallas as pl

B, N = 64, 24
NF, EF = 64, 4
HID, MSG, PASSES = 64, 32, 3

Q = 4
G = B // Q
GB = 4


def _bd_mask(rows, cols, rblk, cblk):
    r = jax.lax.broadcasted_iota(jnp.int32, (rows, cols), 0)
    c = jax.lax.broadcasted_iota(jnp.int32, (rows, cols), 1)
    return (r // rblk == c // cblk).astype(jnp.float32)


def _mpnn_kernel(x_ref, e_ref, W_msg_ref, b_msg_ref, W_u_ref, W_m_ref,
                 b_u_ref, W_g_ref, W_o_ref, out_ref):
    f32 = jnp.float32
    NN = N * N
    QM, QH = Q * MSG, Q * HID
    S0 = _bd_mask(N, NN, 1, N)
    r2_i = jax.lax.broadcasted_iota(jnp.int32, (NN, N), 0)
    g_i = jax.lax.broadcasted_iota(jnp.int32, (NN, N), 1)
    TileG = (r2_i % N == g_i).astype(f32)
    EyeQH = (jax.lax.broadcasted_iota(jnp.int32, (QH, QH), 0) ==
             jax.lax.broadcasted_iota(jnp.int32, (QH, QH), 1)).astype(f32)
    BlkOnes = _bd_mask(Q, QH, 1, HID)
    Ones1N = jnp.ones((1, N), f32)

    W_h = W_msg_ref[0:HID, :]
    W_e = W_msg_ref[HID:HID + EF, :]
    b_msg = b_msg_ref[...]
    W_u = W_u_ref[...]
    W_m = W_m_ref[...]
    b_u = b_u_ref[...]
    W_gh = W_g_ref[0:HID, :]
    W_gx = W_g_ref[HID:2 * HID, :]
    W_oh = W_o_ref[0:HID, :]
    W_ox = W_o_ref[HID:2 * HID, :]

    WkH = jnp.tile(W_h, (Q, Q)) * _bd_mask(QH, QM, HID, MSG)
    WkE = jnp.tile(W_e, (Q, Q)) * _bd_mask(Q * EF, QM, EF, MSG)
    A32 = jnp.tile(jnp.ones((EF, MSG), f32), (Q, Q)) * _bd_mask(
        Q * EF, QM, EF, MSG)
    WkU = jnp.tile(W_u, (Q, Q)) * _bd_mask(QH, QH, HID, HID)
    WkM = jnp.tile(W_m, (Q, Q)) * _bd_mask(QM, QH, MSG, HID)
    WkUM = jnp.concatenate([WkU, WkM], axis=0)
    m64 = _bd_mask(QH, QH, HID, HID)
    WkG = jnp.concatenate([jnp.tile(W_gh, (Q, Q)) * m64,
                           jnp.tile(W_gx, (Q, Q)) * m64], axis=0)
    WkO = jnp.concatenate([jnp.tile(W_oh, (Q, Q)) * m64,
                           jnp.tile(W_ox, (Q, Q)) * m64], axis=0)
    b_msg4 = jnp.tile(b_msg, (1, Q))
    b_u4 = jnp.tile(b_u, (1, Q))

    dot = functools.partial(jnp.dot, preferred_element_type=f32)

    xs4, E4s, mask4s, act4s = [], [], [], []
    for gb in range(GB):
        x4 = jnp.zeros((N, QH), f32)
        E4 = jnp.broadcast_to(b_msg4, (NN, QM))
        adj32 = jnp.zeros((NN, QM), f32)
        act64 = jnp.zeros((N, QH), f32)
        for q in range(Q):
            e_q = e_ref[gb * Q + q]
            x_q = x_ref[gb * Q + q]
            x4 = x4 + dot(x_q, EyeQH[q * HID:(q + 1) * HID, :])
            E4 = E4 + dot(e_q, WkE[q * EF:(q + 1) * EF, :])
            adj32 = adj32 + dot(e_q, A32[q * EF:(q + 1) * EF, :])
            asum_q = dot(S0, jnp.sum(e_q, axis=1, keepdims=True))
            act64 = act64 + dot(asum_q, BlkOnes[q:q + 1, :])
        xs4.append(x4)
        E4s.append(E4)
        mask4s.append((adj32 != 0.0).astype(f32))
        act4s.append(act64 != 0.0)

    hs = list(xs4)
    for _ in range(PASSES):
        Hms = [dot(h, WkH) for h in hs]
        Ts = [jnp.tanh(E4s[i] + dot(TileG, Hms[i]))
              for i in range(GB)]
        msgs = [dot(S0, Ts[i] * mask4s[i]) for i in range(GB)]
        hs = [jnp.where(
            act4s[i],
            jnp.tanh(dot(jnp.concatenate([hs[i], msgs[i]], axis=1),
                         WkUM) + b_u4),
            hs[i]) for i in range(GB)]

    for gb in range(GB):
        hx = jnp.concatenate([hs[gb], xs4[gb]], axis=1)
        gate = jax.nn.sigmoid(dot(hx, WkG))
        o = dot(hx, WkO)
        gated = gate * o * act4s[gb].astype(f32)
        out_ref[gb] = dot(Ones1N, gated)


def kernel(nodes, edges, W_msg, b_msg, W_u, W_m, b_u, W_g, W_o):
    f32 = jnp.float32
    eflat = edges.reshape(B, N * N, EF)
    b_msg2 = b_msg.reshape(1, MSG)
    b_u2 = b_u.reshape(1, HID)

    full = lambda shape: pl.BlockSpec(shape, lambda i: (0,) * len(shape))
    out = pl.pallas_call(
        _mpnn_kernel,
        grid=(G // GB,),
        in_specs=[
            pl.BlockSpec((GB * Q, N, NF), lambda i: (i, 0, 0)),
            pl.BlockSpec((GB * Q, N * N, EF), lambda i: (i, 0, 0)),
            full((HID + EF, MSG)),
            full((1, MSG)),
            full((HID, HID)),
            full((MSG, HID)),
            full((1, HID)),
            full((2 * HID, HID)),
            full((2 * HID, HID)),
        ],
        out_specs=pl.BlockSpec((GB, 1, Q * HID), lambda i: (i, 0, 0)),
        out_shape=jax.ShapeDtypeStruct((G, 1, Q * HID), f32),
    )(nodes, eflat, W_msg, b_msg2, W_u, W_m, b_u2, W_g, W_o)
    return out.reshape(B, HID)


# speedup vs baseline: 1.7832x; 1.2787x over previous
import jax
import jax.numpy as jnp
from jax.experimental import pallas as pl

B, N = 64, 24
NF, EF = 64, 4
HID, MSG, PASSES = 64, 32, 3

Q = 4
G = B // Q
GB = 4

_DT = (((0,), (0,)), ((), ()))


def _dgT(a, b):
    return jax.lax.dot_general(a, b, _DT, preferred_element_type=jnp.float32)


def _dot(a, b):
    return jnp.dot(a, b, preferred_element_type=jnp.float32)


def _bd_mask(rows, cols, rblk, cblk):
    r = jax.lax.broadcasted_iota(jnp.int32, (rows, cols), 0)
    c = jax.lax.broadcasted_iota(jnp.int32, (rows, cols), 1)
    return (r // rblk == c // cblk).astype(jnp.float32)


def _mpnn_kernel(xt_ref, et_ref, W_msg_ref, b_msg_ref, W_u_ref, W_m_ref,
                 b_u_ref, W_g_ref, W_o_ref, out_ref):
    f32 = jnp.float32
    NN = N * N
    QM, QH = Q * MSG, Q * HID
    QE = Q * EF
    S0t = _bd_mask(NN, N, N, 1)
    g_i = jax.lax.broadcasted_iota(jnp.int32, (N, NN), 0)
    r_i = jax.lax.broadcasted_iota(jnp.int32, (N, NN), 1)
    TileGt = (r_i % N == g_i).astype(f32)
    BlkSum = _bd_mask(Q, QE, 1, EF)
    BlkOnes64 = _bd_mask(QH, Q, HID, 1)
    A32t = _bd_mask(QM, QE, MSG, EF)
    OnesN1 = jnp.ones((N, 1), f32)
    Ones11 = jnp.ones((1, 1), f32)

    W_h = W_msg_ref[0:HID, :]
    W_e = W_msg_ref[HID:HID + EF, :]
    b_msg = b_msg_ref[...]
    W_u = W_u_ref[...]
    W_m = W_m_ref[...]
    b_u = b_u_ref[...]
    W_gh = W_g_ref[0:HID, :]
    W_gx = W_g_ref[HID:2 * HID, :]
    W_oh = W_o_ref[0:HID, :]
    W_ox = W_o_ref[HID:2 * HID, :]

    WkH = jnp.tile(W_h, (Q, Q)) * _bd_mask(QH, QM, HID, MSG)
    WkE = jnp.tile(W_e, (Q, Q)) * _bd_mask(QE, QM, EF, MSG)
    WkU = jnp.tile(W_u, (Q, Q)) * _bd_mask(QH, QH, HID, HID)
    WkM = jnp.tile(W_m, (Q, Q)) * _bd_mask(QM, QH, MSG, HID)
    WkUM = jnp.concatenate([WkU, WkM], axis=0)
    m64 = _bd_mask(QH, QH, HID, HID)
    WkG = jnp.concatenate([jnp.tile(W_gh, (Q, Q)) * m64,
                           jnp.tile(W_gx, (Q, Q)) * m64], axis=0)
    WkO = jnp.concatenate([jnp.tile(W_oh, (Q, Q)) * m64,
                           jnp.tile(W_ox, (Q, Q)) * m64], axis=0)
    b_msg4t = _dgT(jnp.tile(b_msg, (1, Q)), Ones11)
    b_u4t = _dgT(jnp.tile(b_u, (1, Q)), Ones11)

    ets = [et_ref[gb] for gb in range(GB)]
    xts = [xt_ref[gb] for gb in range(GB)]
    E4s = [_dgT(WkE, et) + b_msg4t for et in ets]
    mask4s = [(_dot(A32t, et) != 0.0).astype(f32) for et in ets]
    gsums = [_dot(et, S0t) for et in ets]
    act4s = [_dot(BlkOnes64, _dot(BlkSum, gs)) != 0.0 for gs in gsums]

    hs = list(xts)
    for _ in range(PASSES):
        Hms = [_dgT(WkH, h) for h in hs]
        Ts = [jnp.tanh(E4s[i] + _dot(Hms[i], TileGt))
              for i in range(GB)]
        msgs = [_dot(Ts[i] * mask4s[i], S0t) for i in range(GB)]
        hs = [jnp.where(
            act4s[i],
            jnp.tanh(_dgT(WkUM,
                          jnp.concatenate([hs[i], msgs[i]], axis=0))
                     + b_u4t),
            hs[i]) for i in range(GB)]

    for gb in range(GB):
        hx = jnp.concatenate([hs[gb], xts[gb]], axis=0)
        gate = jax.nn.sigmoid(_dgT(WkG, hx))
        o = _dgT(WkO, hx)
        gated = gate * o * act4s[gb].astype(f32)
        out_ref[gb] = _dot(gated, OnesN1)


def kernel(nodes, edges, W_msg, b_msg, W_u, W_m, b_u, W_g, W_o):
    f32 = jnp.float32
    et = edges.reshape(G, Q, N * N, EF).transpose(0, 1, 3, 2) \
              .reshape(G, Q * EF, N * N)
    xt = nodes.reshape(G, Q, N, NF).transpose(0, 1, 3, 2) \
              .reshape(G, Q * NF, N)
    b_msg2 = b_msg.reshape(1, MSG)
    b_u2 = b_u.reshape(1, HID)

    full = lambda shape: pl.BlockSpec(shape, lambda i: (0,) * len(shape))
    out = pl.pallas_call(
        _mpnn_kernel,
        grid=(G // GB,),
        in_specs=[
            pl.BlockSpec((GB, Q * NF, N), lambda i: (i, 0, 0)),
            pl.BlockSpec((GB, Q * EF, N * N), lambda i: (i, 0, 0)),
            full((HID + EF, MSG)),
            full((1, MSG)),
            full((HID, HID)),
            full((MSG, HID)),
            full((1, HID)),
            full((2 * HID, HID)),
            full((2 * HID, HID)),
        ],
        out_specs=pl.BlockSpec((GB, Q * HID, 1), lambda i: (i, 0, 0)),
        out_shape=jax.ShapeDtypeStruct((G, Q * HID, 1), f32),
    )(xt, et, W_msg, b_msg2, W_u, W_m, b_u2, W_g, W_o)
    return out.reshape(B, HID)


# speedup vs baseline: 1.8797x; 1.0541x over previous
import jax
import jax.numpy as jnp
from jax.experimental import pallas as pl

B, N = 64, 24
NF, EF = 64, 4
HID, MSG, PASSES = 64, 32, 3

Q = 4
G = B // Q
GB = 4

_DT = (((0,), (0,)), ((), ()))


def _dgT(a, b):
    return jax.lax.dot_general(a, b, _DT, preferred_element_type=jnp.float32)


def _dot(a, b):
    return jnp.dot(a, b, preferred_element_type=jnp.float32)


def _bd_mask(rows, cols, rblk, cblk):
    r = jax.lax.broadcasted_iota(jnp.int32, (rows, cols), 0)
    c = jax.lax.broadcasted_iota(jnp.int32, (rows, cols), 1)
    return (r // rblk == c // cblk).astype(jnp.float32)


def _mpnn_kernel(x_ref, et_ref, W_msg_ref, b_msg_ref, W_u_ref, W_m_ref,
                 b_u_ref, W_g_ref, W_o_ref, out_ref):
    f32 = jnp.float32
    NN = N * N
    QM, QH = Q * MSG, Q * HID
    QE = Q * EF
    S0t = _bd_mask(NN, N, N, 1)
    g_i = jax.lax.broadcasted_iota(jnp.int32, (N, NN), 0)
    r_i = jax.lax.broadcasted_iota(jnp.int32, (N, NN), 1)
    TileGt = (r_i % N == g_i).astype(f32)
    BlkSum = _bd_mask(Q, QE, 1, EF)
    BlkOnes64 = _bd_mask(QH, Q, HID, 1)
    A32t = _bd_mask(QM, QE, MSG, EF)
    OnesN1 = jnp.ones((N, 1), f32)
    Ones11 = jnp.ones((1, 1), f32)
    EyeN = (jax.lax.broadcasted_iota(jnp.int32, (N, N), 0) ==
            jax.lax.broadcasted_iota(jnp.int32, (N, N), 1)).astype(f32)

    W_h = W_msg_ref[0:HID, :]
    W_e = W_msg_ref[HID:HID + EF, :]
    b_msg = b_msg_ref[...]
    W_u = W_u_ref[...]
    W_m = W_m_ref[...]
    b_u = b_u_ref[...]
    W_gh = W_g_ref[0:HID, :]
    W_gx = W_g_ref[HID:2 * HID, :]
    W_oh = W_o_ref[0:HID, :]
    W_ox = W_o_ref[HID:2 * HID, :]

    WkH = jnp.tile(W_h, (Q, Q)) * _bd_mask(QH, QM, HID, MSG)
    WkE = jnp.tile(W_e, (Q, Q)) * _bd_mask(QE, QM, EF, MSG)
    WkU = jnp.tile(W_u, (Q, Q)) * _bd_mask(QH, QH, HID, HID)
    WkM = jnp.tile(W_m, (Q, Q)) * _bd_mask(QM, QH, MSG, HID)
    WkUM = jnp.concatenate([WkU, WkM], axis=0)
    m64 = _bd_mask(QH, QH, HID, HID)
    WkG = jnp.concatenate([jnp.tile(W_gh, (Q, Q)) * m64,
                           jnp.tile(W_gx, (Q, Q)) * m64], axis=0)
    WkO = jnp.concatenate([jnp.tile(W_oh, (Q, Q)) * m64,
                           jnp.tile(W_ox, (Q, Q)) * m64], axis=0)
    b_msg4t = _dgT(jnp.tile(b_msg, (1, Q)), Ones11)
    b_u4t = _dgT(jnp.tile(b_u, (1, Q)), Ones11)

    ets = [et_ref[gb] for gb in range(GB)]
    xts = [jnp.concatenate(
        [_dgT(x_ref[gb * Q + q], EyeN) for q in range(Q)], axis=0)
        for gb in range(GB)]
    E4s = [_dgT(WkE, et) + b_msg4t for et in ets]
    mask4s = [(_dot(A32t, et) != 0.0).astype(f32) for et in ets]
    gsums = [_dot(et, S0t) for et in ets]
    act4s = [_dot(BlkOnes64, _dot(BlkSum, gs)) != 0.0 for gs in gsums]

    hs = list(xts)
    for _ in range(PASSES):
        Hms = [_dgT(WkH, h) for h in hs]
        Ts = [jnp.tanh(E4s[i] + _dot(Hms[i], TileGt))
              for i in range(GB)]
        msgs = [_dot(Ts[i] * mask4s[i], S0t) for i in range(GB)]
        hs = [jnp.where(
            act4s[i],
            jnp.tanh(_dgT(WkUM,
                          jnp.concatenate([hs[i], msgs[i]], axis=0))
                     + b_u4t),
            hs[i]) for i in range(GB)]

    for gb in range(GB):
        hx = jnp.concatenate([hs[gb], xts[gb]], axis=0)
        gate = jax.nn.sigmoid(_dgT(WkG, hx))
        o = _dgT(WkO, hx)
        gated = gate * o * act4s[gb].astype(f32)
        out_ref[gb] = _dot(gated, OnesN1)


def kernel(nodes, edges, W_msg, b_msg, W_u, W_m, b_u, W_g, W_o):
    f32 = jnp.float32
    et = edges.reshape(G, Q, N * N, EF).transpose(0, 1, 3, 2) \
              .reshape(G, Q * EF, N * N)
    b_msg2 = b_msg.reshape(1, MSG)
    b_u2 = b_u.reshape(1, HID)

    full = lambda shape: pl.BlockSpec(shape, lambda i: (0,) * len(shape))
    out = pl.pallas_call(
        _mpnn_kernel,
        grid=(G // GB,),
        in_specs=[
            pl.BlockSpec((GB * Q, N, NF), lambda i: (i, 0, 0)),
            pl.BlockSpec((GB, Q * EF, N * N), lambda i: (i, 0, 0)),
            full((HID + EF, MSG)),
            full((1, MSG)),
            full((HID, HID)),
            full((MSG, HID)),
            full((1, HID)),
            full((2 * HID, HID)),
            full((2 * HID, HID)),
        ],
        out_specs=pl.BlockSpec((GB, Q * HID, 1), lambda i: (i, 0, 0)),
        out_shape=jax.ShapeDtypeStruct((G, Q * HID, 1), f32),
    )(nodes, et, W_msg, b_msg2, W_u, W_m, b_u2, W_g, W_o)
    return out.reshape(B, HID)


# speedup vs baseline: 1.9676x; 1.0468x over previous
import jax
import jax.numpy as jnp
from jax.experimental import pallas as pl

B, N = 64, 24
NF, EF = 64, 4
HID, MSG, PASSES = 64, 32, 3

Q = 4
G = B // Q
GB = 8

_DT = (((0,), (0,)), ((), ()))


def _dgT(a, b):
    return jax.lax.dot_general(a, b, _DT, preferred_element_type=jnp.float32)


def _dot(a, b):
    return jnp.dot(a, b, preferred_element_type=jnp.float32)


def _bd_mask(rows, cols, rblk, cblk):
    r = jax.lax.broadcasted_iota(jnp.int32, (rows, cols), 0)
    c = jax.lax.broadcasted_iota(jnp.int32, (rows, cols), 1)
    return (r // rblk == c // cblk).astype(jnp.float32)


def _mpnn_kernel(x_ref, et_ref, W_msg_ref, b_msg_ref, W_u_ref, W_m_ref,
                 b_u_ref, W_g_ref, W_o_ref, out_ref):
    f32 = jnp.float32
    NN = N * N
    QM, QH = Q * MSG, Q * HID
    QE = Q * EF
    S0t = _bd_mask(NN, N, N, 1)
    g_i = jax.lax.broadcasted_iota(jnp.int32, (N, NN), 0)
    r_i = jax.lax.broadcasted_iota(jnp.int32, (N, NN), 1)
    TileGt = (r_i % N == g_i).astype(f32)
    BlkSum = _bd_mask(Q, QE, 1, EF)
    BlkOnes64 = _bd_mask(QH, Q, HID, 1)
    A32t = _bd_mask(QM, QE, MSG, EF)
    OnesN1 = jnp.ones((N, 1), f32)
    Ones11 = jnp.ones((1, 1), f32)
    EyeN = (jax.lax.broadcasted_iota(jnp.int32, (N, N), 0) ==
            jax.lax.broadcasted_iota(jnp.int32, (N, N), 1)).astype(f32)

    W_h = W_msg_ref[0:HID, :]
    W_e = W_msg_ref[HID:HID + EF, :]
    b_msg = b_msg_ref[...]
    W_u = W_u_ref[...]
    W_m = W_m_ref[...]
    b_u = b_u_ref[...]
    W_gh = W_g_ref[0:HID, :]
    W_gx = W_g_ref[HID:2 * HID, :]
    W_oh = W_o_ref[0:HID, :]
    W_ox = W_o_ref[HID:2 * HID, :]

    WkH = jnp.tile(W_h, (Q, Q)) * _bd_mask(QH, QM, HID, MSG)
    WkE = jnp.tile(W_e, (Q, Q)) * _bd_mask(QE, QM, EF, MSG)
    WkU = jnp.tile(W_u, (Q, Q)) * _bd_mask(QH, QH, HID, HID)
    WkM = jnp.tile(W_m, (Q, Q)) * _bd_mask(QM, QH, MSG, HID)
    WkUM = jnp.concatenate([WkU, WkM], axis=0)
    m64 = _bd_mask(QH, QH, HID, HID)
    WkG = jnp.concatenate([jnp.tile(W_gh, (Q, Q)) * m64,
                           jnp.tile(W_gx, (Q, Q)) * m64], axis=0)
    WkO = jnp.concatenate([jnp.tile(W_oh, (Q, Q)) * m64,
                           jnp.tile(W_ox, (Q, Q)) * m64], axis=0)
    b_msg4t = _dgT(jnp.tile(b_msg, (1, Q)), Ones11)
    b_u4t = _dgT(jnp.tile(b_u, (1, Q)), Ones11)

    ets = [et_ref[gb] for gb in range(GB)]
    xts = [jnp.concatenate(
        [_dgT(x_ref[gb * Q + q], EyeN) for q in range(Q)], axis=0)
        for gb in range(GB)]
    E4s = [_dgT(WkE, et) + b_msg4t for et in ets]
    mask4s = [(_dot(A32t, et) != 0.0).astype(f32) for et in ets]
    gsums = [_dot(et, S0t) for et in ets]
    act4s = [_dot(BlkOnes64, _dot(BlkSum, gs)) != 0.0 for gs in gsums]

    hs = list(xts)
    for _ in range(PASSES):
        Hms = [_dgT(WkH, h) for h in hs]
        Ts = [jnp.tanh(E4s[i] + _dot(Hms[i], TileGt))
              for i in range(GB)]
        msgs = [_dot(Ts[i] * mask4s[i], S0t) for i in range(GB)]
        hs = [jnp.where(
            act4s[i],
            jnp.tanh(_dgT(WkUM,
                          jnp.concatenate([hs[i], msgs[i]], axis=0))
                     + b_u4t),
            hs[i]) for i in range(GB)]

    for gb in range(GB):
        hx = jnp.concatenate([hs[gb], xts[gb]], axis=0)
        gate = jax.nn.sigmoid(_dgT(WkG, hx))
        o = _dgT(WkO, hx)
        gated = gate * o * act4s[gb].astype(f32)
        out_ref[gb] = _dot(gated, OnesN1)


def kernel(nodes, edges, W_msg, b_msg, W_u, W_m, b_u, W_g, W_o):
    f32 = jnp.float32
    et = edges.reshape(G, Q, N * N, EF).transpose(0, 1, 3, 2) \
              .reshape(G, Q * EF, N * N)
    b_msg2 = b_msg.reshape(1, MSG)
    b_u2 = b_u.reshape(1, HID)

    full = lambda shape: pl.BlockSpec(shape, lambda i: (0,) * len(shape))
    out = pl.pallas_call(
        _mpnn_kernel,
        grid=(G // GB,),
        in_specs=[
            pl.BlockSpec((GB * Q, N, NF), lambda i: (i, 0, 0)),
            pl.BlockSpec((GB, Q * EF, N * N), lambda i: (i, 0, 0)),
            full((HID + EF, MSG)),
            full((1, MSG)),
            full((HID, HID)),
            full((MSG, HID)),
            full((1, HID)),
            full((2 * HID, HID)),
            full((2 * HID, HID)),
        ],
        out_specs=pl.BlockSpec((GB, Q * HID, 1), lambda i: (i, 0, 0)),
        out_shape=jax.ShapeDtypeStruct((G, Q * HID, 1), f32),
    )(nodes, et, W_msg, b_msg2, W_u, W_m, b_u2, W_g, W_o)
    return out.reshape(B, HID)
